# Initial kernel scaffold; baseline (speedup 1.0000x reference)
import jax
import jax.numpy as jnp
from jax.experimental import pallas as pl


def kernel(texts, table, W1, b1, W2, b2):
    raise NotImplementedError("write your pallas kernel here")


# speedup vs baseline: 4.2507x; 4.2507x over previous
import functools

import jax
import jax.numpy as jnp
from jax import lax
from jax.experimental import pallas as pl
from jax.experimental.pallas import tpu as pltpu
from jax.experimental.pallas import tpu_sc as plsc

VOCAB = 100000
EMB = 64
FIX = 50
B = 4096
HIDDEN = 128
OUT = 2

NC = 2
NS = 16
NW = NC * NS

ROWS_W = B // NW
SB = 8
N_SB = ROWS_W // SB


def _make_sc_gather():
    mesh = plsc.VectorSubcoreMesh(core_axis_name="c", subcore_axis_name="s")

    @functools.partial(
        pl.kernel,
        mesh=mesh,
        out_type=jax.ShapeDtypeStruct((B, FIX, EMB), jnp.float32),
        scratch_types=[
            pltpu.VMEM((ROWS_W, FIX), jnp.int32),
            pltpu.VMEM((SB, FIX, EMB), jnp.float32),
            pltpu.SemaphoreType.DMA,
        ],
        compiler_params=pltpu.CompilerParams(use_tc_tiling_on_sc=False),
    )
    def gather_k(table_hbm, idx_hbm, out_hbm, idx_v, stage_v, gsem):
        wid = lax.axis_index("s") * NC + lax.axis_index("c")
        r0 = wid * ROWS_W
        pltpu.sync_copy(idx_hbm.at[pl.ds(r0, ROWS_W)], idx_v)

        def body(s, carry):
            descs = [
                pltpu.async_copy(
                    table_hbm.at[idx_v.at[s * SB + k]], stage_v.at[k], gsem
                )
                for k in range(SB)
            ]
            for d in descs:
                d.wait()
            pltpu.sync_copy(stage_v, out_hbm.at[pl.ds(r0 + s * SB, SB)])
            return carry

        lax.fori_loop(0, N_SB, body, 0)

    return gather_k


_sc_gather = _make_sc_gather()


BM = 256


def _mlp_body(x_ref, w1_ref, b1_ref, w2_ref, b2_ref, o_ref):
    x = x_ref[...].reshape(BM, FIX * EMB)
    h = jnp.dot(x, w1_ref[...], preferred_element_type=jnp.float32) + b1_ref[...]
    h = jnp.where(h >= 0, h, 0.01 * h)
    o_ref[...] = jnp.dot(h, w2_ref[...], preferred_element_type=jnp.float32) + b2_ref[...]


def _tc_mlp(rows, W1, b1, W2, b2):
    K = FIX * EMB
    return pl.pallas_call(
        _mlp_body,
        grid=(B // BM,),
        in_specs=[
            pl.BlockSpec((BM, FIX, EMB), lambda i: (i, 0, 0)),
            pl.BlockSpec((K, HIDDEN), lambda i: (0, 0)),
            pl.BlockSpec((1, HIDDEN), lambda i: (0, 0)),
            pl.BlockSpec((HIDDEN, OUT), lambda i: (0, 0)),
            pl.BlockSpec((1, OUT), lambda i: (0, 0)),
        ],
        out_specs=pl.BlockSpec((BM, OUT), lambda i: (i, 0)),
        out_shape=jax.ShapeDtypeStruct((B, OUT), jnp.float32),
    )(rows, W1, b1.reshape(1, HIDDEN), W2, b2.reshape(1, OUT))


def kernel(texts, table, W1, b1, W2, b2):
    idx = texts.astype(jnp.int32)
    rows = _sc_gather(table, idx)
    return _tc_mlp(rows, W1, b1, W2, b2)


# speedup vs baseline: 6.6147x; 1.5562x over previous
import functools

import jax
import jax.numpy as jnp
from jax import lax
from jax.experimental import pallas as pl
from jax.experimental.pallas import tpu as pltpu
from jax.experimental.pallas import tpu_sc as plsc

VOCAB = 100000
EMB = 64
FIX = 50
B = 4096
HIDDEN = 128
OUT = 2

NC = 2
NS = 16
NW = NC * NS

N_IDX = B * FIX
PER_W = N_IDX // NW
SBI = 400
N_SB = PER_W // SBI
CH = 80
N_CH = SBI // CH
OROWS_W = PER_W * EMB // 128
OROWS_SB = SBI * EMB // 128


def _make_sc_gather():
    mesh = plsc.VectorSubcoreMesh(core_axis_name="c", subcore_axis_name="s")

    @functools.partial(
        pl.kernel,
        mesh=mesh,
        out_type=jax.ShapeDtypeStruct((N_IDX, EMB), jnp.float32),
        scratch_types=[
            pltpu.VMEM((PER_W,), jnp.int32),
            pltpu.VMEM((SBI, EMB), jnp.float32),
            pltpu.SemaphoreType.DMA,
        ],
        compiler_params=pltpu.CompilerParams(use_tc_tiling_on_sc=False),
    )
    def gather_k(table_hbm, idx_hbm, out_hbm, idx_v, stage_v, gsem):
        wid = lax.axis_index("s") * NC + lax.axis_index("c")
        pltpu.sync_copy(idx_hbm.at[pl.ds(wid * PER_W, PER_W)], idx_v)

        def body(s, carry):
            descs = [
                pltpu.async_copy(
                    table_hbm.at[idx_v.at[pl.ds(s * SBI + j * CH, CH)]],
                    stage_v.at[pl.ds(j * CH, CH)],
                    gsem,
                )
                for j in range(N_CH)
            ]
            for d in descs:
                d.wait()
            pltpu.sync_copy(
                stage_v,
                out_hbm.at[pl.ds(wid * PER_W + s * SBI, SBI)],
            )
            return carry

        lax.fori_loop(0, N_SB, body, 0)

    return gather_k


_sc_gather = _make_sc_gather()


BM = 256
XROWS = BM * FIX * EMB // 128


def _mlp_body(x_ref, w1_ref, b1_ref, w2_ref, b2_ref, o_ref):
    x = x_ref[...].reshape(BM, FIX * EMB)
    h = jnp.dot(x, w1_ref[...], preferred_element_type=jnp.float32) + b1_ref[...]
    h = jnp.where(h >= 0, h, 0.01 * h)
    o_ref[...] = jnp.dot(h, w2_ref[...], preferred_element_type=jnp.float32) + b2_ref[...]


def _tc_mlp(rows, W1, b1, W2, b2):
    K = FIX * EMB
    return pl.pallas_call(
        _mlp_body,
        grid=(B // BM,),
        in_specs=[
            pl.BlockSpec((XROWS, 128), lambda i: (i, 0)),
            pl.BlockSpec((K, HIDDEN), lambda i: (0, 0)),
            pl.BlockSpec((1, HIDDEN), lambda i: (0, 0)),
            pl.BlockSpec((HIDDEN, OUT), lambda i: (0, 0)),
            pl.BlockSpec((1, OUT), lambda i: (0, 0)),
        ],
        out_specs=pl.BlockSpec((BM, OUT), lambda i: (i, 0)),
        out_shape=jax.ShapeDtypeStruct((B, OUT), jnp.float32),
    )(rows, W1, b1.reshape(1, HIDDEN), W2, b2.reshape(1, OUT))


def kernel(texts, table, W1, b1, W2, b2):
    idx = texts.reshape(N_IDX).astype(jnp.int32)
    rows = _sc_gather(table, idx)
    rows128 = rows.reshape(N_IDX * EMB // 128, 128)
    return _tc_mlp(rows128, W1, b1, W2, b2)
